# baseline (speedup 1.0000x reference)
import functools

import jax
import jax.numpy as jnp
from jax import lax
from jax.experimental import pallas as pl
from jax.experimental.pallas import tpu as pltpu
from jax.experimental.pallas import tpu_sc as plsc

NUM_CORES = 2
NUM_SUBCORES = 16
NUM_WORKERS = NUM_CORES * NUM_SUBCORES


@functools.partial(jax.jit, static_argnums=(2, 3, 4))
def _gather_rows_packed(idx, table_packed, B, D, chunk):
    D2 = D // 2
    n16 = D2 // 16
    b_per_w = B // NUM_WORKERS
    n_chunks = b_per_w // chunk
    assert n_chunks >= 2
    mesh = plsc.VectorSubcoreMesh(core_axis_name="c", subcore_axis_name="s")

    @functools.partial(
        pl.kernel,
        mesh=mesh,
        out_type=jax.ShapeDtypeStruct((B, D), jnp.float32),
        scratch_types=[
            pltpu.VMEM((b_per_w,), jnp.int32),
            pltpu.VMEM((2, chunk, D2), jnp.int32),
            pltpu.VMEM((2, chunk, D), jnp.float32),
        ]
        + [pltpu.SemaphoreType.DMA] * 4,
    )
    def k(idx_hbm, table_hbm, out_hbm, idx_v, gbufs, fbufs, *sems):
        gsems, ssems = sems[:2], sems[2:]
        wid = lax.axis_index("s") * NUM_CORES + lax.axis_index("c")
        base = wid * b_per_w

        pltpu.sync_copy(idx_hbm.at[pl.ds(base, b_per_w)], idx_v)

        def gather_desc(i, b):
            return pltpu.make_async_copy(
                table_hbm.at[idx_v.at[pl.ds(i * chunk, chunk)]],
                gbufs.at[b],
                gsems[b],
            )

        def store_desc(i, b):
            return pltpu.make_async_copy(
                fbufs.at[b],
                out_hbm.at[pl.ds(base + i * chunk, chunk)],
                ssems[b],
            )

        hi_mask = jnp.full((16,), -65536, jnp.int32)

        def convert_chunk(b):
            gb = gbufs.at[b]
            fb = fbufs.at[b]

            def row_body(r, carry):
                for m in range(n16):
                    w = gb[r, pl.ds(m * 16, 16)]
                    lo = lax.bitcast_convert_type(
                        lax.shift_left(w, 16), jnp.float32)
                    hi = lax.bitcast_convert_type(
                        lax.bitwise_and(w, hi_mask), jnp.float32)
                    fb[r, pl.ds(m * 16, 16)] = lo
                    fb[r, pl.ds(D2 + m * 16, 16)] = hi
                return carry

            lax.fori_loop(0, chunk, row_body, 0)

        gather_desc(0, 0).start()

        def body(i, carry):
            b = lax.rem(i, 2)

            @pl.when(b == 0)
            def _():
                _pipe_step(i, 0)

            @pl.when(b == 1)
            def _():
                _pipe_step(i, 1)

            return carry

        def _pipe_step(i, b):
            gather_desc(i, b).wait()

            @pl.when(i + 1 < n_chunks)
            def _():
                gather_desc(i + 1, 1 - b).start()

            @pl.when(i >= 2)
            def _():
                store_desc(i - 2, b).wait()

            convert_chunk(b)
            store_desc(i, b).start()

        lax.fori_loop(0, n_chunks, body, 0)
        store_desc(n_chunks - 2, n_chunks % 2).wait()
        store_desc(n_chunks - 1, 1 - (n_chunks % 2)).wait()

    return k(idx, table_packed)


def _pack_table(pos_embed):
    D2 = pos_embed.shape[1] // 2
    bits = lax.bitcast_convert_type(
        pos_embed.astype(jnp.bfloat16), jnp.uint16).astype(jnp.uint32)
    packed = bits[:, :D2] | (bits[:, D2:] << 16)
    return lax.bitcast_convert_type(packed, jnp.int32)


def kernel(position_ids, pos_embed):
    b, s = position_ids.shape
    d = pos_embed.shape[1]
    B = b * s
    idx = position_ids.reshape(B).astype(jnp.int32)
    out = _gather_rows_packed(idx, _pack_table(pos_embed), B, d, 32)
    return out.reshape(b, s, d)


# speedup vs baseline: 1.2373x; 1.2373x over previous
import functools

import jax
import jax.numpy as jnp
from jax import lax
from jax.experimental import pallas as pl
from jax.experimental.pallas import tpu as pltpu
from jax.experimental.pallas import tpu_sc as plsc

NUM_CORES = 2
NUM_SUBCORES = 16
NUM_WORKERS = NUM_CORES * NUM_SUBCORES


@functools.partial(jax.jit, static_argnums=(2, 3, 4))
def _gather_rows_packed(idx, table_packed, B, D, chunk):
    D2 = D // 2
    n16 = D2 // 16
    b_per_w = B // NUM_WORKERS
    n_chunks = b_per_w // chunk
    assert n_chunks >= 2
    mesh = plsc.VectorSubcoreMesh(core_axis_name="c", subcore_axis_name="s")

    @functools.partial(
        pl.kernel,
        mesh=mesh,
        out_type=jax.ShapeDtypeStruct((B, D), jnp.float32),
        scratch_types=[
            pltpu.VMEM((b_per_w,), jnp.int32),
            pltpu.VMEM((2, chunk, D2), jnp.int32),
            pltpu.VMEM((2, chunk, D), jnp.float32),
        ]
        + [pltpu.SemaphoreType.DMA] * 4,
    )
    def k(idx_hbm, table_hbm, out_hbm, idx_v, gbufs, fbufs, *sems):
        gsems, ssems = sems[:2], sems[2:]
        wid = lax.axis_index("s") * NUM_CORES + lax.axis_index("c")
        base = wid * b_per_w

        pltpu.sync_copy(idx_hbm.at[pl.ds(base, b_per_w)], idx_v)

        def gather_desc(i, b):
            return pltpu.make_async_copy(
                table_hbm.at[idx_v.at[pl.ds(i * chunk, chunk)]],
                gbufs.at[b],
                gsems[b],
            )

        def store_desc(i, b):
            return pltpu.make_async_copy(
                fbufs.at[b],
                out_hbm.at[pl.ds(base + i * chunk, chunk)],
                ssems[b],
            )

        hi_mask = jnp.full((16,), -65536, jnp.int32)

        def convert_chunk(b):
            gb = gbufs.at[b]
            fb = fbufs.at[b]

            for r in range(chunk):
                for m in range(n16):
                    w = gb[r, pl.ds(m * 16, 16)]
                    lo = lax.bitcast_convert_type(
                        lax.shift_left(w, 16), jnp.float32)
                    hi = lax.bitcast_convert_type(
                        lax.bitwise_and(w, hi_mask), jnp.float32)
                    fb[r, pl.ds(m * 16, 16)] = lo
                    fb[r, pl.ds(D2 + m * 16, 16)] = hi

        gather_desc(0, 0).start()

        def body(i, carry):
            b = lax.rem(i, 2)

            @pl.when(b == 0)
            def _():
                _pipe_step(i, 0)

            @pl.when(b == 1)
            def _():
                _pipe_step(i, 1)

            return carry

        def _pipe_step(i, b):
            gather_desc(i, b).wait()

            @pl.when(i + 1 < n_chunks)
            def _():
                gather_desc(i + 1, 1 - b).start()

            @pl.when(i >= 2)
            def _():
                store_desc(i - 2, b).wait()

            convert_chunk(b)
            store_desc(i, b).start()

        lax.fori_loop(0, n_chunks, body, 0)
        store_desc(n_chunks - 2, n_chunks % 2).wait()
        store_desc(n_chunks - 1, 1 - (n_chunks % 2)).wait()

    return k(idx, table_packed)


def _pack_table(pos_embed):
    D2 = pos_embed.shape[1] // 2
    bits = lax.bitcast_convert_type(
        pos_embed.astype(jnp.bfloat16), jnp.uint16).astype(jnp.uint32)
    packed = bits[:, :D2] | (bits[:, D2:] << 16)
    return lax.bitcast_convert_type(packed, jnp.int32)


def kernel(position_ids, pos_embed):
    b, s = position_ids.shape
    d = pos_embed.shape[1]
    B = b * s
    idx = position_ids.reshape(B).astype(jnp.int32)
    out = _gather_rows_packed(idx, _pack_table(pos_embed), B, d, 32)
    return out.reshape(b, s, d)


# speedup vs baseline: 1.4924x; 1.2061x over previous
import functools

import jax
import jax.numpy as jnp
from jax import lax
from jax.experimental import pallas as pl
from jax.experimental.pallas import tpu as pltpu
from jax.experimental.pallas import tpu_sc as plsc

NUM_CORES = 2
NUM_SUBCORES = 16
NUM_WORKERS = NUM_CORES * NUM_SUBCORES


NBUF = 4
LOOKAHEAD = 2


@functools.partial(jax.jit, static_argnums=(2, 3, 4))
def _gather_rows(idx, table, B, D, chunk):
    b_per_w = B // NUM_WORKERS
    n_chunks = b_per_w // chunk
    assert n_chunks >= NBUF and n_chunks % NBUF == 0
    mesh = plsc.VectorSubcoreMesh(core_axis_name="c", subcore_axis_name="s")

    @functools.partial(
        pl.kernel,
        mesh=mesh,
        out_type=jax.ShapeDtypeStruct((B, D), jnp.float32),
        scratch_types=[
            pltpu.VMEM((b_per_w,), jnp.int32),
            pltpu.VMEM((NBUF, chunk, D), jnp.float32),
        ]
        + [pltpu.SemaphoreType.DMA] * (2 * NBUF),
    )
    def k(idx_hbm, table_hbm, out_hbm, idx_v, bufs, *sems):
        gsems, ssems = sems[:NBUF], sems[NBUF:]
        wid = lax.axis_index("s") * NUM_CORES + lax.axis_index("c")
        base = wid * b_per_w

        pltpu.sync_copy(idx_hbm.at[pl.ds(base, b_per_w)], idx_v)

        def gather_desc(i, b):
            return pltpu.make_async_copy(
                table_hbm.at[idx_v.at[pl.ds(i * chunk, chunk)]],
                bufs.at[b],
                gsems[b],
            )

        def store_desc(i, b):
            return pltpu.make_async_copy(
                bufs.at[b],
                out_hbm.at[pl.ds(base + i * chunk, chunk)],
                ssems[b],
            )

        for j in range(LOOKAHEAD):
            gather_desc(j, j).start()

        def body(it, carry):
            g = it * NBUF
            for b in range(NBUF):
                i = g + b
                gather_desc(i, b).wait()
                store_desc(i, b).start()

                j = i + LOOKAHEAD
                bj = (b + LOOKAHEAD) % NBUF

                @pl.when(j < n_chunks)
                def _():
                    @pl.when(j >= NBUF)
                    def _():
                        store_desc(j - NBUF, bj).wait()

                    gather_desc(j, bj).start()

            return carry

        lax.fori_loop(0, n_chunks // NBUF, body, 0)
        for b in range(NBUF):
            i_last = n_chunks - NBUF + b
            store_desc(i_last, (i_last % NBUF)).wait()

    return k(idx, table)


def kernel(position_ids, pos_embed):
    b, s = position_ids.shape
    d = pos_embed.shape[1]
    idx = position_ids.reshape(b * s).astype(jnp.int32)
    out = _gather_rows(idx, pos_embed, b * s, d, 32)
    return out.reshape(b, s, d)
